# baseline (speedup 1.0000x reference)
import jax
import jax.numpy as jnp
from jax import lax
from jax.experimental import pallas as pl
from jax.experimental.pallas import tpu as pltpu
from jax.experimental.pallas import tpu_sc as plsc

B = 4096
S = 200
D = 64
H = 256
C = 2

NC = 2
NS = 16
NW = NC * NS
BPW = B // NW

_S0 = 128
_S1 = S - _S0


def _sc_pool_body(x_hbm, emb_hbm, out_hbm, idx_v, rows0, rows1, rows2, rows3,
                  rows4, rows5, out_v, sem0, sem1, sem2, sem3, sem4, sem5):
    wid = lax.axis_index("s") * NC + lax.axis_index("c")
    base = wid * BPW

    pltpu.sync_copy(x_hbm.at[pl.ds(base, BPW)], idx_v)

    def fire(b, buf, sem):
        pltpu.async_copy(emb_hbm.at[idx_v.at[b, pl.ds(0, _S0)]],
                         buf.at[pl.ds(0, _S0)], sem)
        pltpu.async_copy(emb_hbm.at[idx_v.at[b, pl.ds(_S0, _S1)]],
                         buf.at[pl.ds(_S0, _S1)], sem)

    def drain(b, buf, sem):
        pltpu.make_async_copy(emb_hbm.at[idx_v.at[b, pl.ds(0, _S0)]],
                              buf.at[pl.ds(0, _S0)], sem).wait()
        pltpu.make_async_copy(emb_hbm.at[idx_v.at[b, pl.ds(_S0, _S1)]],
                              buf.at[pl.ds(_S0, _S1)], sem).wait()

    def process(b, buf, sem):
        drain(b, buf, sem)
        zero = jnp.zeros((16,), jnp.float32)

        def acc_body(j, accs):
            return tuple(a + buf[j, pl.ds(16 * k, 16)] for k, a in enumerate(accs))

        a0, a1, a2, a3 = lax.fori_loop(0, S, acc_body, (zero, zero, zero, zero))

        out_v[b, pl.ds(0, 16)] = a0
        out_v[b, pl.ds(16, 16)] = a1
        out_v[b, pl.ds(32, 16)] = a2
        out_v[b, pl.ds(48, 16)] = a3

    bufs = ((rows0, sem0), (rows1, sem1), (rows2, sem2), (rows3, sem3),
            (rows4, sem4), (rows5, sem5))
    nb = len(bufs)
    for p, (buf, sem) in enumerate(bufs):
        fire(p, buf, sem)

    def loop_body(g, carry):
        b = nb * g
        for p, (buf, sem) in enumerate(bufs):
            process(b + p, buf, sem)

            @pl.when(b + p + nb < BPW)
            def _():
                fire(b + p + nb, buf, sem)

        return carry

    lax.fori_loop(0, BPW // nb, loop_body, jnp.int32(0))
    process(126, rows0, sem0)
    process(127, rows1, sem1)

    pltpu.sync_copy(out_v, out_hbm.at[pl.ds(base, BPW)])


@jax.jit
def _sc_pool(x, emb):
    mesh = plsc.VectorSubcoreMesh(core_axis_name="c", subcore_axis_name="s")
    return pl.kernel(
        _sc_pool_body,
        out_type=jax.ShapeDtypeStruct((B, D), jnp.float32),
        mesh=mesh,
        scratch_types=[
            pltpu.VMEM((BPW, S), jnp.int32),
            pltpu.VMEM((S, D), jnp.float32),
            pltpu.VMEM((S, D), jnp.float32),
            pltpu.VMEM((S, D), jnp.float32),
            pltpu.VMEM((S, D), jnp.float32),
            pltpu.VMEM((S, D), jnp.float32),
            pltpu.VMEM((S, D), jnp.float32),
            pltpu.VMEM((BPW, D), jnp.float32),
            pltpu.SemaphoreType.DMA,
            pltpu.SemaphoreType.DMA,
            pltpu.SemaphoreType.DMA,
            pltpu.SemaphoreType.DMA,
            pltpu.SemaphoreType.DMA,
            pltpu.SemaphoreType.DMA,
        ],
        compiler_params=pltpu.CompilerParams(use_tc_tiling_on_sc=False),
    )(x, emb)


def _mlp_body(x_ref, summed_ref, W1_ref, b1_ref, W2_ref, b2_ref, out_ref):
    denom = jnp.sum((x_ref[...] != 0).astype(jnp.float32), axis=1, keepdims=True)
    avg = summed_ref[...] / jnp.maximum(denom, 1.0)
    h = jnp.dot(avg, W1_ref[...], preferred_element_type=jnp.float32)
    h = jnp.maximum(h + b1_ref[...], 0.0)
    logits = jnp.dot(h, W2_ref[...], preferred_element_type=jnp.float32)
    logits = logits + b2_ref[...]
    m = jnp.max(logits, axis=1, keepdims=True)
    s = logits - m
    lse = jnp.log(jnp.sum(jnp.exp(s), axis=1, keepdims=True))
    out_ref[...] = s - lse


@jax.jit
def _tc_mlp(x, summed, W1, b1, W2, b2):
    return pl.pallas_call(
        _mlp_body,
        out_shape=jax.ShapeDtypeStruct((B, C), jnp.float32),
    )(x, summed, W1, b1.reshape(1, H), W2, b2.reshape(1, C))


def kernel(x, emb, W1, b1, W2, b2):
    summed = _sc_pool(x, emb)
    return _tc_mlp(x, summed, W1, b1, W2, b2)
